# Initial kernel scaffold; baseline (speedup 1.0000x reference)
import jax
import jax.numpy as jnp
from jax.experimental import pallas as pl


def kernel(c_seq, d_seq, r_seq, X, v_r, v_beta, W_ih, W_hh, b_ih, b_hh, W1a, b1a, W1b, b1b, W2a, b2a, W2b, b2b):
    raise NotImplementedError("write your pallas kernel here")


# speedup vs baseline: 3.0168x; 3.0168x over previous
import functools

import jax
import jax.numpy as jnp
from jax.experimental import pallas as pl
from jax.experimental.pallas import tpu as pltpu

NUM_C = 512
NUM_D = 8
DV = 64
B = 16
T = 200
BT = B * T

_HIGH = jax.lax.Precision.HIGHEST


def _main_body(x_idx_smem, c_smem, d_smem, r_vmem, X_ref, vr_ref, vbeta_ref,
               WihT_ref, bih_ref, WhhT_ref, bhh_ref,
               W2aLT_ref, W2aRT_ref, b2a_ref, W2bT_ref, b2b_ref,
               h_out, c_out,
               gin_ref, gx_ref, base_ref, u_ref, state_ref, h_ref):
    t = pl.program_id(0)

    @pl.when(t == 0)
    def _setup():
        state_ref[...] = jnp.zeros_like(state_ref)
        h_ref[...] = jnp.zeros_like(h_ref)
        u_ref[...] = jax.lax.dot_general(
            vbeta_ref[...], W2aLT_ref[...], (((1,), (0,)), ((), ())),
            precision=_HIGH)
        gin_ref[:, DV:] = r_vmem[...] * vr_ref[...]

        def gather_one(i, _):
            idx = x_idx_smem[i // B, i % B]
            gin_ref[pl.ds(i, 1), 0:DV] = X_ref[pl.ds(idx, 1), :]
            return 0

        jax.lax.fori_loop(0, BT, gather_one, 0, unroll=8)
        gx_ref[...] = jax.lax.dot_general(
            gin_ref[...], WihT_ref[...], (((1,), (0,)), ((), ())),
            precision=_HIGH) + bih_ref[...]
        base_ref[...] = jax.lax.dot_general(
            gin_ref[...], W2aRT_ref[...], (((1,), (0,)), ((), ())),
            precision=_HIGH) + b2a_ref[...]

    @pl.when(t > 0)
    def _step():
        t0 = t - 1
        h = h_ref[...]
        gh = jax.lax.dot_general(h, WhhT_ref[...], (((1,), (0,)), ((), ())),
                                 precision=_HIGH) + bhh_ref[...]
        gx = gx_ref[pl.ds(t0 * B, B), :]
        r_g = jax.nn.sigmoid(gx[:, 0:DV] + gh[:, 0:DV])
        z_g = jax.nn.sigmoid(gx[:, DV:2 * DV] + gh[:, DV:2 * DV])
        n_g = jnp.tanh(gx[:, 2 * DV:] + r_g * gh[:, 2 * DV:])
        h_new = (1.0 - z_g) * n_g + z_g * h
        h_ref[...] = h_new
        h_out[0] = h_new

        lane_iota = jax.lax.broadcasted_iota(jnp.int32, (1, 128), 1)
        betas = []
        for b in range(B):
            c_b = c_smem[t0, b]
            d_b = d_smem[t0, b]
            row = state_ref[b, pl.ds(c_b // 16, 1), :]
            lane = (c_b % 16) * 8 + d_b
            sel = (lane_iota == lane).astype(jnp.float32)
            betas.append(jnp.sum(row * sel, axis=1, keepdims=True))
        beta_col = jnp.concatenate(betas, axis=0)

        pre = base_ref[pl.ds(t0 * B, B), :] + beta_col * u_ref[...]
        act = jnp.maximum(pre, 0.0)
        new_c = jax.lax.dot_general(
            act, W2bT_ref[...], (((1,), (0,)), ((), ())),
            precision=_HIGH) + b2b_ref[...]

        iota8 = jax.lax.broadcasted_iota(jnp.int32, (NUM_D, 128), 0)
        lane8 = jax.lax.broadcasted_iota(jnp.int32, (NUM_D, 128), 1)
        for b in range(B):
            c_b = c_smem[t0, b]
            lane0 = (c_b % 16) * 8
            P = (lane8 == (lane0 + iota8)).astype(jnp.float32)
            placed = jax.lax.dot_general(
                new_c[b:b + 1, :], P, (((1,), (0,)), ((), ())),
                precision=_HIGH)
            row = state_ref[b, pl.ds(c_b // 16, 1), :]
            mask = (lane_iota >= lane0) & (lane_iota < lane0 + NUM_D)
            state_ref[b, pl.ds(c_b // 16, 1), :] = jnp.where(mask, placed, row)

        c_out[:, 0] = state_ref[...]


def _alpha_body(h_ref, W1aT_ref, b1a_ref, W1bT_ref, b1b_ref, out_ref):
    h = h_ref[...].reshape(BT, DV)
    a1 = jax.lax.dot_general(h, W1aT_ref[...], (((1,), (0,)), ((), ())),
                             precision=_HIGH) + b1a_ref[...]
    a1 = jnp.maximum(a1, 0.0)
    out_ref[...] = jax.lax.dot_general(
        a1, W1bT_ref[...], (((1,), (0,)), ((), ())),
        precision=_HIGH) + b1b_ref[...]


def kernel(c_seq, d_seq, r_seq, X, v_r, v_beta, W_ih, W_hh, b_ih, b_hh,
           W1a, b1a, W1b, b1b, W2a, b2a, W2b, b2b):
    c_seq = c_seq.astype(jnp.int32)
    d_seq = d_seq.astype(jnp.int32)
    x_idx_T = (c_seq + NUM_C * d_seq).T
    c_T = c_seq.T
    d_T = d_seq.T
    r_T = r_seq.T.reshape(BT, 1)

    smem = pl.BlockSpec(memory_space=pltpu.MemorySpace.SMEM)
    anyv = pl.BlockSpec(memory_space=pltpu.MemorySpace.VMEM)

    grid_spec = pltpu.PrefetchScalarGridSpec(
        num_scalar_prefetch=0,
        grid=(T + 1,),
        in_specs=[smem, smem, smem] + [anyv] * 13,
        out_specs=[
            pl.BlockSpec((1, B, DV), lambda t: (jnp.maximum(t - 1, 0), 0, 0)),
            pl.BlockSpec((B, 1, 32, 128),
                         lambda t: (0, jnp.maximum(t - 1, 0), 0, 0)),
        ],
        scratch_shapes=[
            pltpu.VMEM((BT, 2 * DV), jnp.float32),
            pltpu.VMEM((BT, 3 * DV), jnp.float32),
            pltpu.VMEM((BT, DV), jnp.float32),
            pltpu.VMEM((1, DV), jnp.float32),
            pltpu.VMEM((B, 32, 128), jnp.float32),
            pltpu.VMEM((B, DV), jnp.float32),
        ],
    )
    h_seq_t, c_seq4 = pl.pallas_call(
        _main_body,
        grid_spec=grid_spec,
        out_shape=[
            jax.ShapeDtypeStruct((T, B, DV), jnp.float32),
            jax.ShapeDtypeStruct((B, T, 32, 128), jnp.float32),
        ],
        compiler_params=pltpu.CompilerParams(
            dimension_semantics=("arbitrary",)),
    )(x_idx_T, c_T, d_T, r_T, X, v_r.reshape(1, DV), v_beta.reshape(1, DV),
      W_ih.T, b_ih.reshape(1, 3 * DV), W_hh.T, b_hh.reshape(1, 3 * DV),
      W2a[:, :DV].T, W2a[:, DV:].T, b2a.reshape(1, DV),
      W2b.T, b2b.reshape(1, NUM_D))

    alpha_t = pl.pallas_call(
        _alpha_body,
        out_shape=jax.ShapeDtypeStruct((BT, NUM_D), jnp.float32),
    )(h_seq_t, W1a.T, b1a.reshape(1, DV), W1b.T, b1b.reshape(1, NUM_D))

    h_seq = jnp.swapaxes(h_seq_t, 0, 1)
    alpha_seq = jnp.swapaxes(alpha_t.reshape(T, B, NUM_D), 0, 1)
    C_seq = c_seq4.reshape(B, T, NUM_C, NUM_D)
    return (alpha_seq, h_seq, C_seq)
